# Initial kernel scaffold; baseline (speedup 1.0000x reference)
import jax
import jax.numpy as jnp
from jax.experimental import pallas as pl


def kernel(output, target):
    raise NotImplementedError("write your pallas kernel here")


# speedup vs baseline: 54.3803x; 54.3803x over previous
import functools

import jax
import jax.numpy as jnp
from jax import lax
from jax.experimental import pallas as pl
from jax.experimental.pallas import tpu as pltpu
from jax.experimental.pallas import tpu_sc as plsc

NB = 2048
SLOT = 2 * NB
NCLS = 20
HSZ = NCLS * SLOT
NW = 32
CH = 8192
RB = 64


def _bucketize_body(x_ref, t_ref, ids_ref):
    x = x_ref[0]
    m = jnp.max(x, axis=0, keepdims=True)
    ex = jnp.exp(x - m)
    p = ex / jnp.sum(ex, axis=0, keepdims=True)
    lab = t_ref[0]
    valid = lab != 0
    cshape = x.shape
    cidx = lax.broadcasted_iota(jnp.int32, cshape, 0)
    fg = (lab[None, :, :] == cidx) & valid[None, :, :]
    e = jnp.abs(fg.astype(jnp.float32) - p)
    b = jnp.minimum((e * NB).astype(jnp.int32), NB - 1)
    idx = cidx * SLOT + jnp.where(fg, NB, 0) + b
    idx = jnp.where(valid[None, :, :], idx, 0)
    ids_ref[:, 0] = idx


def _make_bucketize(Bb, C, H, W):
    return pl.pallas_call(
        _bucketize_body,
        grid=(Bb, H // RB),
        in_specs=[
            pl.BlockSpec((1, C, RB, W), lambda b, r: (b, 0, r, 0)),
            pl.BlockSpec((1, RB, W), lambda b, r: (b, r, 0)),
        ],
        out_specs=pl.BlockSpec((C, 1, RB, W), lambda b, r: (0, b, r, 0)),
        out_shape=jax.ShapeDtypeStruct((C, Bb, H, W), jnp.int32),
    )


def _sc_hist_body(el_per_tile, ids_hbm, out_hbm, hist, buf0, buf1, sem0, sem1):
    wid = lax.axis_index("s") * 2 + lax.axis_index("c")
    base = wid * el_per_tile
    nchunks = el_per_tile // CH

    bufs = (buf0, buf1)
    sems = (sem0, sem1)
    cps = [None, None]
    cps[0] = pltpu.async_copy(ids_hbm.at[pl.ds(base, CH)], buf0, sem0)

    def zero_body(i, _):
        hist[pl.ds(i * 16, 16)] = jnp.zeros((16,), jnp.int32)
        return 0

    lax.fori_loop(0, HSZ // 16, zero_body, 0)

    ones = jnp.ones((16,), jnp.int32)

    for k in range(nchunks):
        cur = k % 2
        cps[cur].wait()
        if k + 1 < nchunks:
            nxt = 1 - cur
            cps[nxt] = pltpu.async_copy(
                ids_hbm.at[pl.ds(base + (k + 1) * CH, CH)], bufs[nxt], sems[nxt])

        def scat_body(j, _):
            iv = bufs[cur][pl.ds(j * 16, 16)]
            plsc.addupdate_scatter(hist, [iv], ones)
            return 0

        lax.fori_loop(0, CH // 16, scat_body, 0)

    pltpu.sync_copy(hist, out_hbm.at[wid])


def _make_sc_hist(cp_total):
    el_per_tile = cp_total // NW
    assert el_per_tile % CH == 0
    mesh = plsc.VectorSubcoreMesh(core_axis_name="c", subcore_axis_name="s")
    return pl.kernel(
        functools.partial(_sc_hist_body, el_per_tile),
        out_type=jax.ShapeDtypeStruct((NW, HSZ), jnp.int32),
        mesh=mesh,
        compiler_params=pltpu.CompilerParams(needs_layout_passes=False),
        scratch_types=[
            pltpu.VMEM((HSZ,), jnp.int32),
            pltpu.VMEM((CH,), jnp.int32),
            pltpu.VMEM((CH,), jnp.int32),
            pltpu.SemaphoreType.DMA,
            pltpu.SemaphoreType.DMA,
        ],
    )


def _finalize_body(p_ref, o_ref):
    ph = p_ref[...]
    h = jnp.sum(ph.reshape(NW, NCLS, SLOT), axis=0)
    hf = h.astype(jnp.float32)
    cnt0 = hf[:, :NB]
    cnt1 = hf[:, NB:]
    mm = cnt0 + cnt1
    tt = cnt1
    rows = lax.broadcasted_iota(jnp.int32, (NB, NB), 0)
    cols = lax.broadcasted_iota(jnp.int32, (NB, NB), 1)
    lt = (rows <= cols).astype(jnp.float32)
    dn = (((1,), (0,)), ((), ()))
    Mf = lax.dot_general(mm, lt, dn, precision=lax.Precision.HIGHEST,
                         preferred_element_type=jnp.float32)
    Kf = lax.dot_general(tt, lt, dn, precision=lax.Precision.HIGHEST,
                         preferred_element_type=jnp.float32)
    T = Mf[:, NB - 1:NB]
    G1 = Kf[:, NB - 1:NB]
    n_lo = T - Mf
    k_lo = G1 - Kf
    n_hi = n_lo + mm
    k_hi = k_lo + tt
    J_lo = 1.0 - (G1 - k_lo) / jnp.maximum(G1 + n_lo - k_lo, 1.0)
    J_hi = 1.0 - (G1 - k_hi) / jnp.maximum(G1 + n_hi - k_hi, 1.0)
    centers = (lax.broadcasted_iota(jnp.int32, (NCLS, NB), 1).astype(jnp.float32)
               + 0.5) * (1.0 / NB)
    contrib = jnp.where(mm > 0, centers * (J_hi - J_lo), 0.0)
    loss = jnp.sum(contrib, axis=1, keepdims=True)
    present = (G1 > 0).astype(jnp.float32)
    res = jnp.sum(loss * present) / jnp.sum(present)
    o_ref[...] = jnp.broadcast_to(res, (1, 1))


def _make_finalize():
    return pl.pallas_call(
        _finalize_body,
        in_specs=[pl.BlockSpec((NW * NCLS, SLOT), lambda: (0, 0))],
        out_specs=pl.BlockSpec((1, 1), lambda: (0, 0)),
        out_shape=jax.ShapeDtypeStruct((1, 1), jnp.float32),
    )


def kernel(output, target):
    Bb, C, H, W = output.shape
    ids = _make_bucketize(Bb, C, H, W)(output, target)
    cp_total = C * Bb * H * W
    partials = _make_sc_hist(cp_total)(ids.reshape(cp_total))
    parts = partials.reshape(NW * NCLS, SLOT)
    return _make_finalize()(parts).reshape(())


# speedup vs baseline: 60.3968x; 1.1106x over previous
import functools

import jax
import jax.numpy as jnp
from jax import lax
from jax.experimental import pallas as pl
from jax.experimental.pallas import tpu as pltpu
from jax.experimental.pallas import tpu_sc as plsc

NB = 2048
SLOT = 2 * NB
NCLS = 20
HSZ = NCLS * SLOT
NW = 32
CH = 8192
RB = 64


def _bucketize_body(x_ref, t_ref, ids_ref):
    x = x_ref[0]
    m = jnp.max(x, axis=0, keepdims=True)
    ex = jnp.exp(x - m)
    p = ex / jnp.sum(ex, axis=0, keepdims=True)
    lab = t_ref[0]
    valid = lab != 0
    cshape = x.shape
    cidx = lax.broadcasted_iota(jnp.int32, cshape, 0)
    fg = (lab[None, :, :] == cidx) & valid[None, :, :]
    e = jnp.abs(fg.astype(jnp.float32) - p)
    b = jnp.minimum((e * NB).astype(jnp.int32), NB - 1)
    idx = cidx * SLOT + jnp.where(fg, NB, 0) + b
    idx = jnp.where(valid[None, :, :], idx, 0)
    ids_ref[:, 0] = idx


def _make_bucketize(Bb, C, H, W):
    return pl.pallas_call(
        _bucketize_body,
        grid=(Bb, H // RB),
        in_specs=[
            pl.BlockSpec((1, C, RB, W), lambda b, r: (b, 0, r, 0)),
            pl.BlockSpec((1, RB, W), lambda b, r: (b, r, 0)),
        ],
        out_specs=pl.BlockSpec((C, 1, RB, W), lambda b, r: (0, b, r, 0)),
        out_shape=jax.ShapeDtypeStruct((C, Bb, H, W), jnp.int32),
    )


def _sc_hist_body(el_per_tile, ids_hbm, out_hbm, hist, buf0, buf1, sem0, sem1):
    wid = lax.axis_index("s") * 2 + lax.axis_index("c")
    base = wid * el_per_tile
    nchunks = el_per_tile // CH

    bufs = (buf0, buf1)
    sems = (sem0, sem1)
    cps = [None, None]
    cps[0] = pltpu.async_copy(ids_hbm.at[pl.ds(base, CH)], buf0, sem0)

    UNZ = 16
    zeros = jnp.zeros((16,), jnp.int32)

    def zero_body(i, _):
        for u in range(UNZ):
            hist[pl.ds(i * (16 * UNZ) + u * 16, 16)] = zeros
        return 0

    lax.fori_loop(0, HSZ // (16 * UNZ), zero_body, 0)

    ones = jnp.ones((16,), jnp.int32)
    UNS = 16

    for k in range(nchunks):
        cur = k % 2
        cps[cur].wait()
        if k + 1 < nchunks:
            nxt = 1 - cur
            cps[nxt] = pltpu.async_copy(
                ids_hbm.at[pl.ds(base + (k + 1) * CH, CH)], bufs[nxt], sems[nxt])

        def scat_body(j, _):
            for u in range(UNS):
                iv = bufs[cur][pl.ds(j * (16 * UNS) + u * 16, 16)]
                plsc.addupdate_scatter(hist, [iv], ones)
            return 0

        lax.fori_loop(0, CH // (16 * UNS), scat_body, 0)

    pltpu.sync_copy(hist, out_hbm.at[wid])


def _make_sc_hist(cp_total):
    el_per_tile = cp_total // NW
    assert el_per_tile % CH == 0
    mesh = plsc.VectorSubcoreMesh(core_axis_name="c", subcore_axis_name="s")
    return pl.kernel(
        functools.partial(_sc_hist_body, el_per_tile),
        out_type=jax.ShapeDtypeStruct((NW, HSZ), jnp.int32),
        mesh=mesh,
        compiler_params=pltpu.CompilerParams(needs_layout_passes=False),
        scratch_types=[
            pltpu.VMEM((HSZ,), jnp.int32),
            pltpu.VMEM((CH,), jnp.int32),
            pltpu.VMEM((CH,), jnp.int32),
            pltpu.SemaphoreType.DMA,
            pltpu.SemaphoreType.DMA,
        ],
    )


def _finalize_body(p_ref, o_ref):
    ph = p_ref[...]
    h = jnp.sum(ph.reshape(NW, NCLS, SLOT), axis=0)
    hf = h.astype(jnp.float32)
    cnt0 = hf[:, :NB]
    cnt1 = hf[:, NB:]
    mm = cnt0 + cnt1
    tt = cnt1
    rows = lax.broadcasted_iota(jnp.int32, (NB, NB), 0)
    cols = lax.broadcasted_iota(jnp.int32, (NB, NB), 1)
    lt = (rows <= cols).astype(jnp.float32)
    dn = (((1,), (0,)), ((), ()))
    Mf = lax.dot_general(mm, lt, dn, precision=lax.Precision.HIGHEST,
                         preferred_element_type=jnp.float32)
    Kf = lax.dot_general(tt, lt, dn, precision=lax.Precision.HIGHEST,
                         preferred_element_type=jnp.float32)
    T = Mf[:, NB - 1:NB]
    G1 = Kf[:, NB - 1:NB]
    n_lo = T - Mf
    k_lo = G1 - Kf
    n_hi = n_lo + mm
    k_hi = k_lo + tt
    J_lo = 1.0 - (G1 - k_lo) / jnp.maximum(G1 + n_lo - k_lo, 1.0)
    J_hi = 1.0 - (G1 - k_hi) / jnp.maximum(G1 + n_hi - k_hi, 1.0)
    centers = (lax.broadcasted_iota(jnp.int32, (NCLS, NB), 1).astype(jnp.float32)
               + 0.5) * (1.0 / NB)
    contrib = jnp.where(mm > 0, centers * (J_hi - J_lo), 0.0)
    loss = jnp.sum(contrib, axis=1, keepdims=True)
    present = (G1 > 0).astype(jnp.float32)
    res = jnp.sum(loss * present) / jnp.sum(present)
    o_ref[...] = jnp.broadcast_to(res, (1, 1))


def _make_finalize():
    return pl.pallas_call(
        _finalize_body,
        in_specs=[pl.BlockSpec((NW * NCLS, SLOT), lambda: (0, 0))],
        out_specs=pl.BlockSpec((1, 1), lambda: (0, 0)),
        out_shape=jax.ShapeDtypeStruct((1, 1), jnp.float32),
    )


def kernel(output, target):
    Bb, C, H, W = output.shape
    ids = _make_bucketize(Bb, C, H, W)(output, target)
    cp_total = C * Bb * H * W
    partials = _make_sc_hist(cp_total)(ids.reshape(cp_total))
    parts = partials.reshape(NW * NCLS, SLOT)
    return _make_finalize()(parts).reshape(())


# speedup vs baseline: 88.2580x; 1.4613x over previous
import functools

import jax
import jax.numpy as jnp
from jax import lax
from jax.experimental import pallas as pl
from jax.experimental.pallas import tpu as pltpu
from jax.experimental.pallas import tpu_sc as plsc

NB = 2048
SLOT = 2 * NB
NCLS = 20
HSZ = NCLS * SLOT
NW = 32
CH = 8192
RB = 64


def _bucketize_body(x_ref, t_ref, ids_ref):
    x = x_ref[0]
    m = jnp.max(x, axis=0, keepdims=True)
    ex = jnp.exp(x - m)
    p = ex / jnp.sum(ex, axis=0, keepdims=True)
    lab = t_ref[0]
    valid = lab != 0
    cshape = x.shape
    cidx = lax.broadcasted_iota(jnp.int32, cshape, 0)
    fg = (lab[None, :, :] == cidx) & valid[None, :, :]
    e = jnp.abs(fg.astype(jnp.float32) - p)
    b = jnp.minimum((e * NB).astype(jnp.int32), NB - 1)
    idx = cidx * SLOT + jnp.where(fg, NB, 0) + b
    idx = jnp.where(valid[None, :, :], idx, 0)
    ids_ref[:, 0] = idx


def _make_bucketize(Bb, C, H, W):
    return pl.pallas_call(
        _bucketize_body,
        grid=(Bb, H // RB),
        in_specs=[
            pl.BlockSpec((1, C, RB, W), lambda b, r: (b, 0, r, 0)),
            pl.BlockSpec((1, RB, W), lambda b, r: (b, r, 0)),
        ],
        out_specs=pl.BlockSpec((C, 1, RB, W), lambda b, r: (0, b, r, 0)),
        out_shape=jax.ShapeDtypeStruct((C, Bb, H, W), jnp.int32),
    )


def _sc_hist_body(el_per_tile, ids_hbm, out_hbm, hist, buf0, buf1, sem0, sem1):
    wid = lax.axis_index("s") * 2 + lax.axis_index("c")
    base = wid * el_per_tile
    nchunks = el_per_tile // CH

    bufs = (buf0, buf1)
    sems = (sem0, sem1)
    cps = [None, None]
    cps[0] = pltpu.async_copy(ids_hbm.at[pl.ds(base, CH)], buf0, sem0)

    UNZ = 16
    zeros = jnp.zeros((16,), jnp.int32)

    def zero_body(i, _):
        for u in range(UNZ):
            hist[pl.ds(i * (16 * UNZ) + u * 16, 16)] = zeros
        return 0

    lax.fori_loop(0, HSZ // (16 * UNZ), zero_body, 0)

    ones = jnp.ones((16,), jnp.int32)
    UNS = 16

    for k in range(nchunks):
        cur = k % 2
        cps[cur].wait()
        if k + 1 < nchunks:
            nxt = 1 - cur
            cps[nxt] = pltpu.async_copy(
                ids_hbm.at[pl.ds(base + (k + 1) * CH, CH)], bufs[nxt], sems[nxt])

        def scat_body(j, _):
            ivs = [bufs[cur][pl.ds(j * (16 * UNS) + u * 16, 16)]
                   for u in range(UNS)]
            for iv in ivs:
                plsc.addupdate_scatter(hist, [iv], ones)
            return 0

        lax.fori_loop(0, CH // (16 * UNS), scat_body, 0)

    pltpu.sync_copy(hist, out_hbm.at[wid])


def _make_sc_hist(cp_total):
    el_per_tile = cp_total // NW
    assert el_per_tile % CH == 0
    mesh = plsc.VectorSubcoreMesh(core_axis_name="c", subcore_axis_name="s")
    return pl.kernel(
        functools.partial(_sc_hist_body, el_per_tile),
        out_type=jax.ShapeDtypeStruct((NW, HSZ), jnp.int32),
        mesh=mesh,
        compiler_params=pltpu.CompilerParams(needs_layout_passes=False),
        scratch_types=[
            pltpu.VMEM((HSZ,), jnp.int32),
            pltpu.VMEM((CH,), jnp.int32),
            pltpu.VMEM((CH,), jnp.int32),
            pltpu.SemaphoreType.DMA,
            pltpu.SemaphoreType.DMA,
        ],
    )


def _finalize_body(p_ref, o_ref):
    ph = p_ref[...]
    h = jnp.sum(ph.reshape(NW, NCLS, SLOT), axis=0)
    hf = h.astype(jnp.float32)
    cnt0 = hf[:, :NB]
    cnt1 = hf[:, NB:]
    mm = cnt0 + cnt1
    tt = cnt1
    rows = lax.broadcasted_iota(jnp.int32, (NB, NB), 0)
    cols = lax.broadcasted_iota(jnp.int32, (NB, NB), 1)
    lt = (rows <= cols).astype(jnp.float32)
    dn = (((1,), (0,)), ((), ()))
    Mf = lax.dot_general(mm, lt, dn, precision=lax.Precision.HIGHEST,
                         preferred_element_type=jnp.float32)
    Kf = lax.dot_general(tt, lt, dn, precision=lax.Precision.HIGHEST,
                         preferred_element_type=jnp.float32)
    T = Mf[:, NB - 1:NB]
    G1 = Kf[:, NB - 1:NB]
    n_lo = T - Mf
    k_lo = G1 - Kf
    n_hi = n_lo + mm
    k_hi = k_lo + tt
    J_lo = 1.0 - (G1 - k_lo) / jnp.maximum(G1 + n_lo - k_lo, 1.0)
    J_hi = 1.0 - (G1 - k_hi) / jnp.maximum(G1 + n_hi - k_hi, 1.0)
    centers = (lax.broadcasted_iota(jnp.int32, (NCLS, NB), 1).astype(jnp.float32)
               + 0.5) * (1.0 / NB)
    contrib = jnp.where(mm > 0, centers * (J_hi - J_lo), 0.0)
    loss = jnp.sum(contrib, axis=1, keepdims=True)
    present = (G1 > 0).astype(jnp.float32)
    res = jnp.sum(loss * present) / jnp.sum(present)
    o_ref[...] = jnp.broadcast_to(res, (1, 1))


def _make_finalize():
    return pl.pallas_call(
        _finalize_body,
        in_specs=[pl.BlockSpec((NW * NCLS, SLOT), lambda: (0, 0))],
        out_specs=pl.BlockSpec((1, 1), lambda: (0, 0)),
        out_shape=jax.ShapeDtypeStruct((1, 1), jnp.float32),
    )


def kernel(output, target):
    Bb, C, H, W = output.shape
    ids = _make_bucketize(Bb, C, H, W)(output, target)
    cp_total = C * Bb * H * W
    partials = _make_sc_hist(cp_total)(ids.reshape(cp_total))
    parts = partials.reshape(NW * NCLS, SLOT)
    return _make_finalize()(parts).reshape(())


# speedup vs baseline: 114.2980x; 1.2950x over previous
import functools

import jax
import jax.numpy as jnp
from jax import lax
from jax.experimental import pallas as pl
from jax.experimental.pallas import tpu as pltpu
from jax.experimental.pallas import tpu_sc as plsc

NB = 2048
SLOT = 2 * NB
NCLS = 20
HSZ = NCLS * SLOT
NW = 32
CH = 8192
RB = 64


def _bucketize_body(x_ref, t_ref, ids_ref):
    x = x_ref[0]
    m = jnp.max(x, axis=0, keepdims=True)
    ex = jnp.exp(x - m)
    p = ex / jnp.sum(ex, axis=0, keepdims=True)
    lab = t_ref[0]
    valid = lab != 0
    cshape = x.shape
    cidx = lax.broadcasted_iota(jnp.int32, cshape, 0)
    fg = (lab[None, :, :] == cidx) & valid[None, :, :]
    e = jnp.abs(fg.astype(jnp.float32) - p)
    b = jnp.minimum((e * NB).astype(jnp.int32), NB - 1)
    idx = cidx * SLOT + jnp.where(fg, NB, 0) + b
    idx = jnp.where(valid[None, :, :], idx, 0)
    C = cshape[0]
    ids_ref[...] = idx.reshape(C * RB, x.shape[2])


def _make_bucketize(Bb, C, H, W):
    hb = H // RB
    return pl.pallas_call(
        _bucketize_body,
        grid=(Bb, hb),
        in_specs=[
            pl.BlockSpec((1, C, RB, W), lambda b, r: (b, 0, r, 0)),
            pl.BlockSpec((1, RB, W), lambda b, r: (b, r, 0)),
        ],
        out_specs=pl.BlockSpec((C * RB, W), lambda b, r: (b * hb + r, 0)),
        out_shape=jax.ShapeDtypeStruct((Bb * hb * C * RB, W), jnp.int32),
    )


CROWS = 16


def _sc_hist_body(rows_per_tile, width, ids_hbm, out_hbm, hist, buf0, buf1,
                  sem0, sem1):
    wid = lax.axis_index("s") * 2 + lax.axis_index("c")
    base = wid * rows_per_tile
    nchunks = rows_per_tile // CROWS

    bufs = (buf0, buf1)
    sems = (sem0, sem1)
    cps = [None, None]
    cps[0] = pltpu.async_copy(ids_hbm.at[pl.ds(base, CROWS)], buf0, sem0)

    UNZ = 16
    zeros = jnp.zeros((16,), jnp.int32)

    def zero_body(i, _):
        for u in range(UNZ):
            hist[pl.ds(i * (16 * UNZ) + u * 16, 16)] = zeros
        return 0

    lax.fori_loop(0, HSZ // (16 * UNZ), zero_body, 0)

    ones = jnp.ones((16,), jnp.int32)

    for k in range(nchunks):
        cur = k % 2
        cps[cur].wait()
        if k + 1 < nchunks:
            nxt = 1 - cur
            cps[nxt] = pltpu.async_copy(
                ids_hbm.at[pl.ds(base + (k + 1) * CROWS, CROWS)],
                bufs[nxt], sems[nxt])

        def scat_body(j, _):
            ivs = [bufs[cur][r, pl.ds(j * 16, 16)] for r in range(CROWS)]
            for iv in ivs:
                plsc.addupdate_scatter(hist, [iv], ones)
            return 0

        lax.fori_loop(0, width // 16, scat_body, 0)

    pltpu.sync_copy(hist, out_hbm.at[wid])


def _make_sc_hist(nrows, width):
    rows_per_tile = nrows // NW
    assert rows_per_tile % CROWS == 0
    mesh = plsc.VectorSubcoreMesh(core_axis_name="c", subcore_axis_name="s")
    return pl.kernel(
        functools.partial(_sc_hist_body, rows_per_tile, width),
        out_type=jax.ShapeDtypeStruct((NW, HSZ), jnp.int32),
        mesh=mesh,
        compiler_params=pltpu.CompilerParams(
            needs_layout_passes=False, use_tc_tiling_on_sc=True),
        scratch_types=[
            pltpu.VMEM((HSZ,), jnp.int32),
            pltpu.VMEM((CROWS, 512), jnp.int32),
            pltpu.VMEM((CROWS, 512), jnp.int32),
            pltpu.SemaphoreType.DMA,
            pltpu.SemaphoreType.DMA,
        ],
    )


def _finalize_body(p_ref, o_ref):
    ph = p_ref[...]
    h = jnp.sum(ph.reshape(NW, NCLS, SLOT), axis=0)
    hf = h.astype(jnp.float32)
    cnt0 = hf[:, :NB]
    cnt1 = hf[:, NB:]
    mm = cnt0 + cnt1
    tt = cnt1
    rows = lax.broadcasted_iota(jnp.int32, (NB, NB), 0)
    cols = lax.broadcasted_iota(jnp.int32, (NB, NB), 1)
    lt = (rows <= cols).astype(jnp.float32)
    dn = (((1,), (0,)), ((), ()))
    Mf = lax.dot_general(mm, lt, dn, precision=lax.Precision.HIGHEST,
                         preferred_element_type=jnp.float32)
    Kf = lax.dot_general(tt, lt, dn, precision=lax.Precision.HIGHEST,
                         preferred_element_type=jnp.float32)
    T = Mf[:, NB - 1:NB]
    G1 = Kf[:, NB - 1:NB]
    n_lo = T - Mf
    k_lo = G1 - Kf
    n_hi = n_lo + mm
    k_hi = k_lo + tt
    J_lo = 1.0 - (G1 - k_lo) / jnp.maximum(G1 + n_lo - k_lo, 1.0)
    J_hi = 1.0 - (G1 - k_hi) / jnp.maximum(G1 + n_hi - k_hi, 1.0)
    centers = (lax.broadcasted_iota(jnp.int32, (NCLS, NB), 1).astype(jnp.float32)
               + 0.5) * (1.0 / NB)
    contrib = jnp.where(mm > 0, centers * (J_hi - J_lo), 0.0)
    loss = jnp.sum(contrib, axis=1, keepdims=True)
    present = (G1 > 0).astype(jnp.float32)
    res = jnp.sum(loss * present) / jnp.sum(present)
    o_ref[...] = jnp.broadcast_to(res, (1, 1))


def _make_finalize():
    return pl.pallas_call(
        _finalize_body,
        in_specs=[pl.BlockSpec((NW * NCLS, SLOT), lambda: (0, 0))],
        out_specs=pl.BlockSpec((1, 1), lambda: (0, 0)),
        out_shape=jax.ShapeDtypeStruct((1, 1), jnp.float32),
    )


def kernel(output, target):
    Bb, C, H, W = output.shape
    ids = _make_bucketize(Bb, C, H, W)(output, target)
    partials = _make_sc_hist(ids.shape[0], W)(ids)
    parts = partials.reshape(NW * NCLS, SLOT)
    return _make_finalize()(parts).reshape(())


# speedup vs baseline: 125.0289x; 1.0939x over previous
import functools

import jax
import jax.numpy as jnp
from jax import lax
from jax.experimental import pallas as pl
from jax.experimental.pallas import tpu as pltpu
from jax.experimental.pallas import tpu_sc as plsc

NB = 1024
SLOT = 2 * NB
NCLS = 20
HSZ = NCLS * SLOT
NW = 32
CH = 8192
RB = 64


def _bucketize_body(x_ref, t_ref, ids_ref):
    x = x_ref[0]
    m = jnp.max(x, axis=0, keepdims=True)
    ex = jnp.exp(x - m)
    p = ex / jnp.sum(ex, axis=0, keepdims=True)
    lab = t_ref[0]
    valid = lab != 0
    cshape = x.shape
    cidx = lax.broadcasted_iota(jnp.int32, cshape, 0)
    fg = (lab[None, :, :] == cidx) & valid[None, :, :]
    e = jnp.abs(fg.astype(jnp.float32) - p)
    b = jnp.minimum((e * NB).astype(jnp.int32), NB - 1)
    idx = cidx * SLOT + jnp.where(fg, NB, 0) + b
    idx = jnp.where(valid[None, :, :], idx, 0)
    C = cshape[0]
    ids_ref[...] = idx.reshape(C * RB, x.shape[2])


def _make_bucketize(Bb, C, H, W):
    hb = H // RB
    return pl.pallas_call(
        _bucketize_body,
        grid=(Bb, hb),
        in_specs=[
            pl.BlockSpec((1, C, RB, W), lambda b, r: (b, 0, r, 0)),
            pl.BlockSpec((1, RB, W), lambda b, r: (b, r, 0)),
        ],
        out_specs=pl.BlockSpec((C * RB, W), lambda b, r: (b * hb + r, 0)),
        out_shape=jax.ShapeDtypeStruct((Bb * hb * C * RB, W), jnp.int32),
    )


CROWS = 16


def _sc_hist_body(rows_per_tile, width, ids_hbm, out_hbm, hist, buf0, buf1,
                  sem0, sem1):
    wid = lax.axis_index("s") * 2 + lax.axis_index("c")
    base = wid * rows_per_tile
    nchunks = rows_per_tile // CROWS

    bufs = (buf0, buf1)
    sems = (sem0, sem1)
    cps = [None, None]
    cps[0] = pltpu.async_copy(ids_hbm.at[pl.ds(base, CROWS)], buf0, sem0)

    UNZ = 16
    zeros = jnp.zeros((16,), jnp.int32)

    def zero_body(i, _):
        for u in range(UNZ):
            hist[pl.ds(i * (16 * UNZ) + u * 16, 16)] = zeros
        return 0

    lax.fori_loop(0, HSZ // (16 * UNZ), zero_body, 0)

    ones = jnp.ones((16,), jnp.int32)

    for k in range(nchunks):
        cur = k % 2
        cps[cur].wait()
        if k + 1 < nchunks:
            nxt = 1 - cur
            cps[nxt] = pltpu.async_copy(
                ids_hbm.at[pl.ds(base + (k + 1) * CROWS, CROWS)],
                bufs[nxt], sems[nxt])

        @plsc.parallel_loop(0, width // 16, unroll=2)
        def scat_body(j):
            ivs = [bufs[cur][r, pl.ds(j * 16, 16)] for r in range(CROWS)]
            for iv in ivs:
                plsc.addupdate_scatter(hist, [iv], ones)

    pltpu.sync_copy(hist, out_hbm.at[wid])


def _make_sc_hist(nrows, width):
    rows_per_tile = nrows // NW
    assert rows_per_tile % CROWS == 0
    mesh = plsc.VectorSubcoreMesh(core_axis_name="c", subcore_axis_name="s")
    return pl.kernel(
        functools.partial(_sc_hist_body, rows_per_tile, width),
        out_type=jax.ShapeDtypeStruct((NW, HSZ), jnp.int32),
        mesh=mesh,
        compiler_params=pltpu.CompilerParams(
            needs_layout_passes=False, use_tc_tiling_on_sc=True),
        scratch_types=[
            pltpu.VMEM((HSZ,), jnp.int32),
            pltpu.VMEM((CROWS, 512), jnp.int32),
            pltpu.VMEM((CROWS, 512), jnp.int32),
            pltpu.SemaphoreType.DMA,
            pltpu.SemaphoreType.DMA,
        ],
    )


def _finalize_body(p_ref, o_ref):
    ph = p_ref[...]
    h = jnp.sum(ph.reshape(NW, NCLS, SLOT), axis=0)
    hf = h.astype(jnp.float32)
    cnt0 = hf[:, :NB]
    cnt1 = hf[:, NB:]
    mm = cnt0 + cnt1
    tt = cnt1
    rows = lax.broadcasted_iota(jnp.int32, (NB, NB), 0)
    cols = lax.broadcasted_iota(jnp.int32, (NB, NB), 1)
    lt = (rows <= cols).astype(jnp.float32)
    dn = (((1,), (0,)), ((), ()))
    Mf = lax.dot_general(mm, lt, dn, precision=lax.Precision.HIGHEST,
                         preferred_element_type=jnp.float32)
    Kf = lax.dot_general(tt, lt, dn, precision=lax.Precision.HIGHEST,
                         preferred_element_type=jnp.float32)
    T = Mf[:, NB - 1:NB]
    G1 = Kf[:, NB - 1:NB]
    n_lo = T - Mf
    k_lo = G1 - Kf
    n_hi = n_lo + mm
    k_hi = k_lo + tt
    J_lo = 1.0 - (G1 - k_lo) / jnp.maximum(G1 + n_lo - k_lo, 1.0)
    J_hi = 1.0 - (G1 - k_hi) / jnp.maximum(G1 + n_hi - k_hi, 1.0)
    centers = (lax.broadcasted_iota(jnp.int32, (NCLS, NB), 1).astype(jnp.float32)
               + 0.5) * (1.0 / NB)
    contrib = jnp.where(mm > 0, centers * (J_hi - J_lo), 0.0)
    loss = jnp.sum(contrib, axis=1, keepdims=True)
    present = (G1 > 0).astype(jnp.float32)
    res = jnp.sum(loss * present) / jnp.sum(present)
    o_ref[...] = jnp.broadcast_to(res, (1, 1))


def _make_finalize():
    return pl.pallas_call(
        _finalize_body,
        in_specs=[pl.BlockSpec((NW * NCLS, SLOT), lambda: (0, 0))],
        out_specs=pl.BlockSpec((1, 1), lambda: (0, 0)),
        out_shape=jax.ShapeDtypeStruct((1, 1), jnp.float32),
    )


def kernel(output, target):
    Bb, C, H, W = output.shape
    ids = _make_bucketize(Bb, C, H, W)(output, target)
    partials = _make_sc_hist(ids.shape[0], W)(ids)
    parts = partials.reshape(NW * NCLS, SLOT)
    return _make_finalize()(parts).reshape(())


# speedup vs baseline: 128.9708x; 1.0315x over previous
import functools

import jax
import jax.numpy as jnp
from jax import lax
from jax.experimental import pallas as pl
from jax.experimental.pallas import tpu as pltpu
from jax.experimental.pallas import tpu_sc as plsc

NB = 1024
SLOT = 2 * NB
NCLS = 20
HSZ = NCLS * SLOT
NW = 32
CH = 8192
RB = 64


SCALE = NB - 0.25


def _bucketize_body(x_ref, t_ref, ids_ref):
    x = x_ref[0]
    m = jnp.max(x, axis=0, keepdims=True)
    ex = jnp.exp(x - m)
    sc = SCALE / jnp.sum(ex, axis=0, keepdims=True)
    u = ex * sc
    lab = t_ref[0]
    valid = lab != 0
    cshape = x.shape
    cidx = lax.broadcasted_iota(jnp.int32, cshape, 0)
    fg = (lab[None, :, :] == cidx) & valid[None, :, :]
    val = jnp.where(fg, (NB + SCALE) - u, u)
    idx = cidx * SLOT + val.astype(jnp.int32)
    idx = jnp.where(valid[None, :, :], idx, 0)
    C = cshape[0]
    ids_ref[...] = idx.reshape(C * RB, x.shape[2])


def _make_bucketize(Bb, C, H, W):
    hb = H // RB
    return pl.pallas_call(
        _bucketize_body,
        grid=(Bb, hb),
        in_specs=[
            pl.BlockSpec((1, C, RB, W), lambda b, r: (b, 0, r, 0)),
            pl.BlockSpec((1, RB, W), lambda b, r: (b, r, 0)),
        ],
        out_specs=pl.BlockSpec((C * RB, W), lambda b, r: (b * hb + r, 0)),
        out_shape=jax.ShapeDtypeStruct((Bb * hb * C * RB, W), jnp.int32),
    )


CROWS = 16


def _sc_hist_body(rows_per_tile, width, ids_hbm, out_hbm, hist, buf0, buf1,
                  sem0, sem1):
    wid = lax.axis_index("s") * 2 + lax.axis_index("c")
    base = wid * rows_per_tile
    nchunks = rows_per_tile // CROWS

    bufs = (buf0, buf1)
    sems = (sem0, sem1)
    cps = [None, None]
    cps[0] = pltpu.async_copy(ids_hbm.at[pl.ds(base, CROWS)], buf0, sem0)

    UNZ = 16
    zeros = jnp.zeros((16,), jnp.int32)

    def zero_body(i, _):
        for u in range(UNZ):
            hist[pl.ds(i * (16 * UNZ) + u * 16, 16)] = zeros
        return 0

    lax.fori_loop(0, HSZ // (16 * UNZ), zero_body, 0)

    ones = jnp.ones((16,), jnp.int32)

    for k in range(nchunks):
        cur = k % 2
        cps[cur].wait()
        if k + 1 < nchunks:
            nxt = 1 - cur
            cps[nxt] = pltpu.async_copy(
                ids_hbm.at[pl.ds(base + (k + 1) * CROWS, CROWS)],
                bufs[nxt], sems[nxt])

        @plsc.parallel_loop(0, width // 16, unroll=2)
        def scat_body(j):
            ivs = [bufs[cur][r, pl.ds(j * 16, 16)] for r in range(CROWS)]
            for iv in ivs:
                plsc.addupdate_scatter(hist, [iv], ones)

    pltpu.sync_copy(hist, out_hbm.at[wid])


def _make_sc_hist(nrows, width):
    rows_per_tile = nrows // NW
    assert rows_per_tile % CROWS == 0
    mesh = plsc.VectorSubcoreMesh(core_axis_name="c", subcore_axis_name="s")
    return pl.kernel(
        functools.partial(_sc_hist_body, rows_per_tile, width),
        out_type=jax.ShapeDtypeStruct((NW, HSZ), jnp.int32),
        mesh=mesh,
        compiler_params=pltpu.CompilerParams(
            needs_layout_passes=False, use_tc_tiling_on_sc=True),
        scratch_types=[
            pltpu.VMEM((HSZ,), jnp.int32),
            pltpu.VMEM((CROWS, 512), jnp.int32),
            pltpu.VMEM((CROWS, 512), jnp.int32),
            pltpu.SemaphoreType.DMA,
            pltpu.SemaphoreType.DMA,
        ],
    )


def _finalize_body(p_ref, o_ref):
    ph = p_ref[...]
    h = jnp.sum(ph.reshape(NW, NCLS, SLOT), axis=0)
    hf = h.astype(jnp.float32)
    cnt0 = hf[:, :NB]
    cnt1 = hf[:, NB:]
    mm = cnt0 + cnt1
    tt = cnt1
    rows = lax.broadcasted_iota(jnp.int32, (NB, NB), 0)
    cols = lax.broadcasted_iota(jnp.int32, (NB, NB), 1)
    lt = (rows <= cols).astype(jnp.float32)
    dn = (((1,), (0,)), ((), ()))
    Mf = lax.dot_general(mm, lt, dn, precision=lax.Precision.HIGHEST,
                         preferred_element_type=jnp.float32)
    Kf = lax.dot_general(tt, lt, dn, precision=lax.Precision.HIGHEST,
                         preferred_element_type=jnp.float32)
    T = Mf[:, NB - 1:NB]
    G1 = Kf[:, NB - 1:NB]
    n_lo = T - Mf
    k_lo = G1 - Kf
    n_hi = n_lo + mm
    k_hi = k_lo + tt
    J_lo = 1.0 - (G1 - k_lo) / jnp.maximum(G1 + n_lo - k_lo, 1.0)
    J_hi = 1.0 - (G1 - k_hi) / jnp.maximum(G1 + n_hi - k_hi, 1.0)
    centers = (lax.broadcasted_iota(jnp.int32, (NCLS, NB), 1).astype(jnp.float32)
               + 0.5) * (1.0 / SCALE)
    contrib = jnp.where(mm > 0, centers * (J_hi - J_lo), 0.0)
    loss = jnp.sum(contrib, axis=1, keepdims=True)
    present = (G1 > 0).astype(jnp.float32)
    res = jnp.sum(loss * present) / jnp.sum(present)
    o_ref[...] = jnp.broadcast_to(res, (1, 1))


def _make_finalize():
    return pl.pallas_call(
        _finalize_body,
        in_specs=[pl.BlockSpec((NW * NCLS, SLOT), lambda: (0, 0))],
        out_specs=pl.BlockSpec((1, 1), lambda: (0, 0)),
        out_shape=jax.ShapeDtypeStruct((1, 1), jnp.float32),
    )


def kernel(output, target):
    Bb, C, H, W = output.shape
    ids = _make_bucketize(Bb, C, H, W)(output, target)
    partials = _make_sc_hist(ids.shape[0], W)(ids)
    parts = partials.reshape(NW * NCLS, SLOT)
    return _make_finalize()(parts).reshape(())


# speedup vs baseline: 161.2082x; 1.2500x over previous
import functools

import jax
import jax.numpy as jnp
from jax import lax
from jax.experimental import pallas as pl
from jax.experimental.pallas import tpu as pltpu
from jax.experimental.pallas import tpu_sc as plsc

NB = 1024
SLOT = 2 * NB
NCLS = 20
HSZ = NCLS * SLOT
NW = 32
CH = 8192
RB = 64


SCALE = NB - 0.25


def _bucketize_body(x_ref, t_ref, ids_ref):
    x = x_ref[0]
    m = jnp.max(x, axis=0, keepdims=True)
    ex = jnp.exp(x - m)
    sc = SCALE / jnp.sum(ex, axis=0, keepdims=True)
    u = ex * sc
    lab = t_ref[0]
    valid = lab != 0
    cshape = x.shape
    cidx = lax.broadcasted_iota(jnp.int32, cshape, 0)
    fg = (lab[None, :, :] == cidx) & valid[None, :, :]
    val = jnp.where(fg, (NB + SCALE) - u, u)
    idx = cidx * SLOT + val.astype(jnp.int32)
    idx = jnp.where(valid[None, :, :], idx, 0)
    C = cshape[0]
    flat = idx.reshape(C * RB, x.shape[2])
    half = (C * RB) // 2
    lo = flat[:half].astype(jnp.uint32)
    hi = flat[half:].astype(jnp.uint32)
    ids_ref[...] = lo | (hi << 16)


def _make_bucketize(Bb, C, H, W):
    hb = H // RB
    return pl.pallas_call(
        _bucketize_body,
        grid=(Bb, hb),
        in_specs=[
            pl.BlockSpec((1, C, RB, W), lambda b, r: (b, 0, r, 0)),
            pl.BlockSpec((1, RB, W), lambda b, r: (b, r, 0)),
        ],
        out_specs=pl.BlockSpec((C * RB // 2, W), lambda b, r: (b * hb + r, 0)),
        out_shape=jax.ShapeDtypeStruct((Bb * hb * C * RB // 2, W), jnp.uint32),
    )


CROWS = 16


def _sc_hist_body(rows_per_tile, width, ids_hbm, out_hbm, hist, buf0, buf1,
                  sem0, sem1):
    wid = lax.axis_index("s") * 2 + lax.axis_index("c")
    base = wid * rows_per_tile
    nchunks = rows_per_tile // CROWS

    bufs = (buf0, buf1)
    sems = (sem0, sem1)
    cps = [None, None]
    cps[0] = pltpu.async_copy(ids_hbm.at[pl.ds(base, CROWS)], buf0, sem0)

    UNZ = 16
    zeros = jnp.zeros((16,), jnp.int32)

    def zero_body(i, _):
        for u in range(UNZ):
            hist[pl.ds(i * (16 * UNZ) + u * 16, 16)] = zeros
        return 0

    lax.fori_loop(0, HSZ // (16 * UNZ), zero_body, 0)

    ones = jnp.ones((16,), jnp.int32)

    for k in range(nchunks):
        cur = k % 2
        cps[cur].wait()
        if k + 1 < nchunks:
            nxt = 1 - cur
            cps[nxt] = pltpu.async_copy(
                ids_hbm.at[pl.ds(base + (k + 1) * CROWS, CROWS)],
                bufs[nxt], sems[nxt])

        @plsc.parallel_loop(0, width // 16, unroll=2)
        def scat_body(j):
            ivs = [bufs[cur][r, pl.ds(j * 16, 16)] for r in range(CROWS)]
            for iv in ivs:
                lo = plsc.bitcast(iv & jnp.uint32(0xFFFF), jnp.int32)
                hi = plsc.bitcast(iv >> jnp.uint32(16), jnp.int32)
                plsc.addupdate_scatter(hist, [lo], ones)
                plsc.addupdate_scatter(hist, [hi], ones)

    pltpu.sync_copy(hist, out_hbm.at[wid])


def _make_sc_hist(nrows, width):
    rows_per_tile = nrows // NW
    assert rows_per_tile % CROWS == 0
    mesh = plsc.VectorSubcoreMesh(core_axis_name="c", subcore_axis_name="s")
    return pl.kernel(
        functools.partial(_sc_hist_body, rows_per_tile, width),
        out_type=jax.ShapeDtypeStruct((NW, HSZ), jnp.int32),
        mesh=mesh,
        compiler_params=pltpu.CompilerParams(
            needs_layout_passes=False, use_tc_tiling_on_sc=True),
        scratch_types=[
            pltpu.VMEM((HSZ,), jnp.int32),
            pltpu.VMEM((CROWS, 512), jnp.uint32),
            pltpu.VMEM((CROWS, 512), jnp.uint32),
            pltpu.SemaphoreType.DMA,
            pltpu.SemaphoreType.DMA,
        ],
    )


def _finalize_body(p_ref, o_ref):
    ph = p_ref[...]
    h = jnp.sum(ph.reshape(NW, NCLS, SLOT), axis=0)
    hf = h.astype(jnp.float32)
    cnt0 = hf[:, :NB]
    cnt1 = hf[:, NB:]
    mm = cnt0 + cnt1
    tt = cnt1
    rows = lax.broadcasted_iota(jnp.int32, (NB, NB), 0)
    cols = lax.broadcasted_iota(jnp.int32, (NB, NB), 1)
    lt = (rows <= cols).astype(jnp.float32)
    dn = (((1,), (0,)), ((), ()))
    Mf = lax.dot_general(mm, lt, dn, precision=lax.Precision.HIGHEST,
                         preferred_element_type=jnp.float32)
    Kf = lax.dot_general(tt, lt, dn, precision=lax.Precision.HIGHEST,
                         preferred_element_type=jnp.float32)
    T = Mf[:, NB - 1:NB]
    G1 = Kf[:, NB - 1:NB]
    n_lo = T - Mf
    k_lo = G1 - Kf
    n_hi = n_lo + mm
    k_hi = k_lo + tt
    J_lo = 1.0 - (G1 - k_lo) / jnp.maximum(G1 + n_lo - k_lo, 1.0)
    J_hi = 1.0 - (G1 - k_hi) / jnp.maximum(G1 + n_hi - k_hi, 1.0)
    centers = (lax.broadcasted_iota(jnp.int32, (NCLS, NB), 1).astype(jnp.float32)
               + 0.5) * (1.0 / SCALE)
    contrib = jnp.where(mm > 0, centers * (J_hi - J_lo), 0.0)
    loss = jnp.sum(contrib, axis=1, keepdims=True)
    present = (G1 > 0).astype(jnp.float32)
    res = jnp.sum(loss * present) / jnp.sum(present)
    o_ref[...] = jnp.broadcast_to(res, (1, 1))


def _make_finalize():
    return pl.pallas_call(
        _finalize_body,
        in_specs=[pl.BlockSpec((NW * NCLS, SLOT), lambda: (0, 0))],
        out_specs=pl.BlockSpec((1, 1), lambda: (0, 0)),
        out_shape=jax.ShapeDtypeStruct((1, 1), jnp.float32),
    )


def kernel(output, target):
    Bb, C, H, W = output.shape
    ids = _make_bucketize(Bb, C, H, W)(output, target)
    partials = _make_sc_hist(ids.shape[0], W)(ids)
    parts = partials.reshape(NW * NCLS, SLOT)
    return _make_finalize()(parts).reshape(())


# speedup vs baseline: 161.4985x; 1.0018x over previous
import functools

import jax
import jax.numpy as jnp
from jax import lax
from jax.experimental import pallas as pl
from jax.experimental.pallas import tpu as pltpu
from jax.experimental.pallas import tpu_sc as plsc

NB = 1024
SLOT = 2 * NB
NCLS = 20
HSZ = NCLS * SLOT
NW = 32
RB = 64


SCALE = NB - 0.25


def _bucketize_body(x_ref, t_ref, ids_ref):
    x = x_ref[0]
    m = jnp.max(x, axis=0, keepdims=True)
    ex = jnp.exp(x - m)
    sc = SCALE / jnp.sum(ex, axis=0, keepdims=True)
    u = ex * sc
    lab = t_ref[0]
    valid = lab != 0
    cshape = x.shape
    cidx = lax.broadcasted_iota(jnp.int32, cshape, 0)
    fg = (lab[None, :, :] == cidx) & valid[None, :, :]
    val = jnp.where(fg, (NB + SCALE) - u, u)
    idx = cidx * SLOT + val.astype(jnp.int32)
    idx = jnp.where(valid[None, :, :], idx, 0)
    C = cshape[0]
    flat = idx.reshape(C * RB, x.shape[2])
    half = (C * RB) // 2
    lo = flat[:half].astype(jnp.uint32)
    hi = flat[half:].astype(jnp.uint32)
    ids_ref[...] = lo | (hi << 16)


def _make_bucketize(Bb, C, H, W):
    hb = H // RB
    return pl.pallas_call(
        _bucketize_body,
        grid=(Bb, hb),
        in_specs=[
            pl.BlockSpec((1, C, RB, W), lambda b, r: (b, 0, r, 0)),
            pl.BlockSpec((1, RB, W), lambda b, r: (b, r, 0)),
        ],
        out_specs=pl.BlockSpec((C * RB // 2, W), lambda b, r: (b * hb + r, 0)),
        out_shape=jax.ShapeDtypeStruct((Bb * hb * C * RB // 2, W), jnp.uint32),
    )


CROWS = 16


def _sc_hist_body(rows_per_tile, width, ids_hbm, out_hbm, hist, buf0, buf1,
                  sem0, sem1):
    wid = lax.axis_index("s") * 2 + lax.axis_index("c")
    base = wid * rows_per_tile
    nchunks = rows_per_tile // CROWS

    bufs = (buf0, buf1)
    sems = (sem0, sem1)
    cps = [None, None]
    cps[0] = pltpu.async_copy(ids_hbm.at[pl.ds(base, CROWS)], buf0, sem0)

    UNZ = 16
    zeros = jnp.zeros((16,), jnp.int32)

    def zero_body(i, _):
        for u in range(UNZ):
            hist[pl.ds(i * (16 * UNZ) + u * 16, 16)] = zeros
        return 0

    lax.fori_loop(0, HSZ // (16 * UNZ), zero_body, 0)

    ones = jnp.ones((16,), jnp.int32)

    for k in range(nchunks):
        cur = k % 2
        cps[cur].wait()
        if k + 1 < nchunks:
            nxt = 1 - cur
            cps[nxt] = pltpu.async_copy(
                ids_hbm.at[pl.ds(base + (k + 1) * CROWS, CROWS)],
                bufs[nxt], sems[nxt])

        @plsc.parallel_loop(0, width // 16, unroll=2)
        def scat_body(j):
            ivs = [bufs[cur][r, pl.ds(j * 16, 16)] for r in range(CROWS)]
            for iv in ivs:
                lo = plsc.bitcast(iv & jnp.uint32(0xFFFF), jnp.int32)
                hi = plsc.bitcast(iv >> jnp.uint32(16), jnp.int32)
                plsc.addupdate_scatter(hist, [lo], ones)
                plsc.addupdate_scatter(hist, [hi], ones)

    pltpu.sync_copy(hist, out_hbm.at[wid])


def _make_sc_hist(nrows, width):
    rows_per_tile = nrows // NW
    assert rows_per_tile % CROWS == 0
    mesh = plsc.VectorSubcoreMesh(core_axis_name="c", subcore_axis_name="s")
    return pl.kernel(
        functools.partial(_sc_hist_body, rows_per_tile, width),
        out_type=jax.ShapeDtypeStruct((NW, HSZ), jnp.int32),
        mesh=mesh,
        compiler_params=pltpu.CompilerParams(
            needs_layout_passes=False, use_tc_tiling_on_sc=True),
        scratch_types=[
            pltpu.VMEM((HSZ,), jnp.int32),
            pltpu.VMEM((CROWS, 512), jnp.uint32),
            pltpu.VMEM((CROWS, 512), jnp.uint32),
            pltpu.SemaphoreType.DMA,
            pltpu.SemaphoreType.DMA,
        ],
    )


def _finalize_body(p_ref, o_ref):
    ph = p_ref[...]
    h = jnp.sum(ph.reshape(NW, NCLS, SLOT), axis=0)
    hf = h.astype(jnp.float32)
    cnt0 = hf[:, :NB]
    cnt1 = hf[:, NB:]
    mm = cnt0 + cnt1
    tt = cnt1
    rows = lax.broadcasted_iota(jnp.int32, (NB, NB), 0)
    cols = lax.broadcasted_iota(jnp.int32, (NB, NB), 1)
    lt = (rows <= cols).astype(jnp.float32)
    dn = (((1,), (0,)), ((), ()))
    Mf = lax.dot_general(mm, lt, dn, precision=lax.Precision.HIGHEST,
                         preferred_element_type=jnp.float32)
    Kf = lax.dot_general(tt, lt, dn, precision=lax.Precision.HIGHEST,
                         preferred_element_type=jnp.float32)
    T = Mf[:, NB - 1:NB]
    G1 = Kf[:, NB - 1:NB]
    n_lo = T - Mf
    k_lo = G1 - Kf
    n_hi = n_lo + mm
    k_hi = k_lo + tt
    J_lo = 1.0 - (G1 - k_lo) / jnp.maximum(G1 + n_lo - k_lo, 1.0)
    J_hi = 1.0 - (G1 - k_hi) / jnp.maximum(G1 + n_hi - k_hi, 1.0)
    centers = (lax.broadcasted_iota(jnp.int32, (NCLS, NB), 1).astype(jnp.float32)
               + 0.5) * (1.0 / SCALE)
    contrib = jnp.where(mm > 0, centers * (J_hi - J_lo), 0.0)
    loss = jnp.sum(contrib, axis=1, keepdims=True)
    present = (G1 > 0).astype(jnp.float32)
    res = jnp.sum(loss * present) / jnp.sum(present)
    o_ref[...] = jnp.broadcast_to(res, (1, 1))


def _make_finalize():
    return pl.pallas_call(
        _finalize_body,
        in_specs=[pl.BlockSpec((NW * NCLS, SLOT), lambda: (0, 0))],
        out_specs=pl.BlockSpec((1, 1), lambda: (0, 0)),
        out_shape=jax.ShapeDtypeStruct((1, 1), jnp.float32),
    )


def kernel(output, target):
    Bb, C, H, W = output.shape
    ids = _make_bucketize(Bb, C, H, W)(output, target)
    partials = _make_sc_hist(ids.shape[0], W)(ids)
    parts = partials.reshape(NW * NCLS, SLOT)
    return _make_finalize()(parts).reshape(())
